# baseline (speedup 1.0000x reference)
import functools

import jax
import jax.numpy as jnp
from jax import lax
from jax.experimental import pallas as pl
from jax.experimental.pallas import tpu as pltpu
from jax.experimental.pallas import tpu_sc as plsc

_MASK_RATIO = 0.5


def _tc_mask_kernel(noise_ref, x_ref, out_ref, *, n, num_masked):
    a = noise_ref[:, 0, :]
    ai = a[:, :, None]
    aj = a[:, None, :]
    bc = a.shape[0]
    ii = lax.broadcasted_iota(jnp.int32, (bc, n, n), 1)
    jj = lax.broadcasted_iota(jnp.int32, (bc, n, n), 2)
    before = (aj < ai) | ((aj == ai) & (jj < ii))
    rank = jnp.sum(before.astype(jnp.int32), axis=2)
    masked = rank < num_masked
    out_ref[...] = jnp.where(masked[:, :, None, None], 0.0, x_ref[...])


def _sc_mask_body(noise_hbm, mask_hbm, row2, mvec, *, n, num_masked,
                  num_cores, lanes):
    wid = lax.axis_index("s") * num_cores + lax.axis_index("c")
    base = wid * n
    pltpu.sync_copy(noise_hbm.at[pl.ds(base, n)], row2.at[pl.ds(0, n)])
    pltpu.sync_copy(noise_hbm.at[pl.ds(base, n)], row2.at[pl.ds(n, n)])
    one = jnp.float32(1.0)
    zero = jnp.float32(0.0)
    for k in range(n // lanes):
        tgt = row2[pl.ds(k * lanes, lanes)]
        rank = jnp.where(row2[pl.ds(k * lanes + 1, lanes)] < tgt, one, zero)
        for s in range(2, n):
            w = row2[pl.ds(k * lanes + s, lanes)]
            rank = rank + jnp.where(w < tgt, one, zero)
        val = jnp.where(rank < jnp.float32(num_masked), one, zero)
        mvec[pl.ds(k * lanes, lanes)] = val.astype(jnp.int32)
    pltpu.sync_copy(mvec, mask_hbm.at[pl.ds(base, n)])


def _make_sc_mask(b, n, num_masked):
    info = plsc.get_sparse_core_info()
    num_cores, num_subcores, lanes = (
        info.num_cores, info.num_subcores, info.num_lanes)
    assert b == num_cores * num_subcores and n % lanes == 0
    mesh = plsc.VectorSubcoreMesh(core_axis_name="c", subcore_axis_name="s")
    return functools.partial(
        pl.kernel,
        out_type=jax.ShapeDtypeStruct((b * n,), jnp.int32),
        mesh=mesh,
        scratch_types=[
            pltpu.VMEM((2 * n,), jnp.float32),
            pltpu.VMEM((n,), jnp.int32),
        ],
    )(functools.partial(
        _sc_mask_body, n=n, num_masked=num_masked,
        num_cores=num_cores, lanes=lanes))


def kernel(x):
    b, n, p, d = x.shape
    num_masked = int(_MASK_RATIO * n)
    bc = 4
    noise = jax.random.uniform(jax.random.key(1), (b, n), dtype=jnp.float32)

    mask_i32 = _make_sc_mask(b, n, num_masked)(noise.reshape(b * n))

    noise3 = noise.reshape(b, 1, n)
    out = pl.pallas_call(
        functools.partial(_tc_mask_kernel, n=n, num_masked=num_masked),
        grid=(b // bc,),
        in_specs=[
            pl.BlockSpec((bc, 1, n), lambda i: (i, 0, 0)),
            pl.BlockSpec((bc, n, p, d), lambda i: (i, 0, 0, 0)),
        ],
        out_specs=pl.BlockSpec((bc, n, p, d), lambda i: (i, 0, 0, 0)),
        out_shape=jax.ShapeDtypeStruct((b, n, p, d), x.dtype),
    )(noise3, x)
    return out, mask_i32.reshape(b, n).astype(bool)
